# Initial kernel scaffold; baseline (speedup 1.0000x reference)
import jax
import jax.numpy as jnp
from jax.experimental import pallas as pl


def kernel(features, edge_index, W1, al1, ar1, gn1_alpha, gn1_gamma, gn1_beta, W2, al2, ar2, gn2_alpha, gn2_gamma, gn2_beta, r1_pw, r1_pb, r1_rw, r1_rb, r2_pw, r2_pb, r2_rw, r2_rb, c1_w, c1_b, c2_w, c2_b):
    raise NotImplementedError("write your pallas kernel here")


# speedup vs baseline: 1.0544x; 1.0544x over previous
import functools

import jax
import jax.numpy as jnp
from jax.experimental import pallas as pl

N = 10000
E = 160000
EPS = 1e-5


def _mm_kernel(x_ref, w_ref, o_ref):
    o_ref[...] = jnp.dot(x_ref[...], w_ref[...], preferred_element_type=jnp.float32)


def _matmul(x, w):
    n, k = x.shape
    k2, m = w.shape
    bn = 1000
    return pl.pallas_call(
        _mm_kernel,
        grid=(n // bn,),
        in_specs=[
            pl.BlockSpec((bn, k), lambda i: (i, 0)),
            pl.BlockSpec((k, m), lambda i: (0, 0)),
        ],
        out_specs=pl.BlockSpec((bn, m), lambda i: (i, 0)),
        out_shape=jax.ShapeDtypeStruct((n, m), jnp.float32),
    )(x, w)


def _gat(x, W, al, ar, heads, src, dst):
    n = x.shape[0]
    feat = _matmul(x, W).reshape(n, heads, -1)
    el = jnp.sum(feat * al[None, :, :], axis=-1)
    er = jnp.sum(feat * ar[None, :, :], axis=-1)
    e = jax.nn.leaky_relu(el[src] + er[dst], 0.2)
    ex = jnp.exp(e)
    denom = jax.ops.segment_sum(ex, dst, num_segments=n)
    raw = jax.ops.segment_sum(feat[src] * ex[:, :, None], dst, num_segments=n)
    return raw / jnp.maximum(denom, 1e-9)[:, :, None]


def _graphnorm(x, a, g, b):
    mean = jnp.mean(x, axis=0, keepdims=True)
    sub = x - a * mean
    var = jnp.mean(sub * sub, axis=0, keepdims=True)
    return g * sub / jnp.sqrt(var + EPS) + b


def _readout(x, pw, pb, rw, rb):
    h = jax.nn.relu(x @ pw + pb)
    pooled = jnp.sum(h, axis=0)
    return (pooled @ rw + rb)[None, :]


def kernel(features, edge_index, W1, al1, ar1, gn1_alpha, gn1_gamma, gn1_beta, W2, al2, ar2, gn2_alpha, gn2_gamma, gn2_beta, r1_pw, r1_pb, r1_rw, r1_rb, r2_pw, r2_pb, r2_rw, r2_rb, c1_w, c1_b, c2_w, c2_b):
    src = edge_index[0]
    dst = edge_index[1]
    h = _gat(features, W1, al1, ar1, 4, src, dst)
    h = _graphnorm(h, gn1_alpha, gn1_gamma, gn1_beta)
    h = jax.nn.leaky_relu(h, 0.01)
    r1 = _readout(jnp.mean(h, axis=1), r1_pw, r1_pb, r1_rw, r1_rb)
    h2 = h.reshape(h.shape[0], -1)
    h2 = _gat(h2, W2, al2, ar2, 2, src, dst)
    h2 = _graphnorm(h2, gn2_alpha, gn2_gamma, gn2_beta)
    h2 = jax.nn.leaky_relu(h2, 0.01)
    r2 = _readout(jnp.mean(h2, axis=1), r2_pw, r2_pb, r2_rw, r2_rb)
    readouts = jnp.concatenate([r1, r2], axis=1)
    hc = jax.nn.relu(readouts @ c1_w + c1_b)
    return hc @ c2_w + c2_b


# speedup vs baseline: 13.7630x; 13.0535x over previous
import functools

import jax
import jax.numpy as jnp
from jax import lax
from jax.experimental import pallas as pl
from jax.experimental.pallas import tpu as pltpu
from jax.experimental.pallas import tpu_sc as plsc

N = 10000
E = 160000
EPS = 1e-5
BN = 1024
NPAD = 10240
CHUNK = 128
NCH = E // CHUNK
NS = 16
RPT = NPAD // NS


def _splat(val):
    return jnp.full((16,), val, jnp.int32)


def _make_sc_edge(H):
    mesh = plsc.VectorSubcoreMesh(core_axis_name="c", subcore_axis_name="s")
    out_type = (
        jax.ShapeDtypeStruct((E, 16), jnp.float32),
        jax.ShapeDtypeStruct((2, NPAD, 16), jnp.float32),
    )
    scratch_types = [
        pltpu.VMEM((CHUNK,), jnp.int32),
        pltpu.VMEM((CHUNK,), jnp.int32),
        pltpu.VMEM((CHUNK, 16), jnp.float32),
        pltpu.VMEM((CHUNK, 16), jnp.float32),
        pltpu.VMEM((CHUNK, 16), jnp.float32),
        pltpu.VMEM_SHARED((NPAD, 16), jnp.float32),
        pltpu.SemaphoreType.DMA,
    ]

    @functools.partial(
        pl.kernel, out_type=out_type, mesh=mesh, scratch_types=scratch_types,
        compiler_params=pltpu.CompilerParams(use_tc_tiling_on_sc=False))
    def sc_edge(ltab, rtab, srcv, dstv, zrows16, ext2, denom,
                srcb, dstb, elrows, errows, exrows, densh, sem):
        c = lax.axis_index("c")
        s = lax.axis_index("s")

        pltpu.sync_copy(zrows16, densh.at[pl.ds(s * RPT, RPT)])
        plsc.subcore_barrier()

        nch2 = (NCH // 2) // NS + jnp.where(s < (NCH // 2) % NS, 1, 0)

        def phase1(i, carry):
            ch = (s + NS * i) * 2 + c
            e0 = ch * CHUNK
            pltpu.sync_copy(srcv.at[pl.ds(e0, CHUNK)], srcb)
            pltpu.sync_copy(dstv.at[pl.ds(e0, CHUNK)], dstb)
            pltpu.async_copy(ltab.at[srcb], elrows, sem).wait()
            pltpu.async_copy(rtab.at[dstb], errows, sem).wait()

            def edge(e, carry2):
                a = elrows[e, pl.ds(0, 16)] + errows[e, pl.ds(0, 16)]
                v = jnp.exp(jnp.where(a > 0, a, 0.2 * a))
                exrows[e, pl.ds(0, 16)] = v
                return carry2
            lax.fori_loop(0, CHUNK, edge, 0)

            pltpu.sync_copy(exrows, densh.at[dstb], add=True)
            pltpu.sync_copy(exrows, ext2.at[pl.ds(e0, CHUNK)])
            return carry
        lax.fori_loop(0, nch2, phase1, 0)
        plsc.subcore_barrier()

        sl = pl.ds(s * RPT, RPT)
        pltpu.sync_copy(densh.at[sl], denom.at[c, sl])

    return sc_edge


def _make_sc_agg(Dg, H):
    GPC = Dg // 2
    mesh = plsc.VectorSubcoreMesh(core_axis_name="c", subcore_axis_name="s")
    out_type = (
        jax.ShapeDtypeStruct((Dg, NPAD, 128), jnp.float32),
    )
    scratch_types = [
        pltpu.VMEM((CHUNK,), jnp.int32),
        pltpu.VMEM((CHUNK,), jnp.int32),
        pltpu.VMEM((CHUNK,), jnp.int32),
        pltpu.VMEM((CHUNK + 16,), jnp.float32),
        pltpu.VMEM((CHUNK, 128), jnp.float32),
        pltpu.VMEM_SHARED((NPAD, 128), jnp.float32),
        pltpu.SemaphoreType.DMA,
    ]

    @functools.partial(pl.kernel, out_type=out_type, mesh=mesh,
                       scratch_types=scratch_types)
    def sc_agg(featv, extt, srcv, dstv, zrows, raw,
               srcb, dstb, gidxb, exv, rows, acc, sem):
        c = lax.axis_index("c")
        s = lax.axis_index("s")
        nch = NCH // NS + jnp.where(s < NCH % NS, 1, 0)

        for gi in range(GPC):
            g = c * GPC + gi
            h = g // 2

            pltpu.sync_copy(zrows, acc.at[pl.ds(s * RPT, RPT)])
            plsc.subcore_barrier()

            def phase2(i, carry):
                e0 = (s + NS * i) * CHUNK
                pltpu.sync_copy(srcv.at[pl.ds(e0, CHUNK)], srcb)
                pltpu.sync_copy(dstv.at[pl.ds(e0, CHUNK)], dstb)
                pltpu.sync_copy(extt.at[h, 0, pl.ds(e0, CHUNK)],
                                exv.at[pl.ds(0, CHUNK)])
                for j in range(CHUNK // 16):
                    sv = srcb[pl.ds(j * 16, 16)]
                    gidxb[pl.ds(j * 16, 16)] = sv * Dg + g
                pltpu.async_copy(featv.at[gidxb], rows, sem).wait()

                def edge(e, carry2):
                    ex_e = exv[pl.ds(e, 16)][0]
                    for j in range(8):
                        sl = pl.ds(j * 16, 16)
                        rows[e, sl] = rows[e, sl] * ex_e
                    return carry2
                lax.fori_loop(0, CHUNK, edge, 0)

                pltpu.sync_copy(rows, acc.at[dstb], add=True)
                return carry
            lax.fori_loop(0, nch, phase2, 0)
            plsc.subcore_barrier()

            sl = pl.ds(s * RPT, RPT)
            pltpu.sync_copy(acc.at[sl], raw.at[g, sl])
            plsc.subcore_barrier()

    return sc_agg


_sc_edge_l1 = _make_sc_edge(4)
_sc_edge_l2 = _make_sc_edge(2)
_sc_agg_l1 = _make_sc_agg(8, 4)
_sc_agg_l2 = _make_sc_agg(4, 2)


def _tc_ext_transpose(ext2):
    BE = 3200

    def body(x_ref, o_ref):
        o_ref[...] = x_ref[...].T.reshape(16, 1, BE)

    return pl.pallas_call(
        body,
        grid=(E // BE,),
        in_specs=[pl.BlockSpec((BE, 16), lambda i: (i, 0))],
        out_specs=pl.BlockSpec((16, 1, BE), lambda i: (0, 0, i)),
        out_shape=jax.ShapeDtypeStruct((16, 1, E), jnp.float32),
    )(ext2)


def _tc_feat(x, W, alf, arf, hsel):
    n, K = x.shape
    D = W.shape[1]
    H = hsel.shape[1]
    bn = 1000

    def body(x_ref, w_ref, alf_ref, arf_ref, hsel_ref, feat_ref, l_ref, r_ref):
        feat = jnp.dot(x_ref[...], w_ref[...], preferred_element_type=jnp.float32)
        feat_ref[...] = feat
        el = jnp.dot(feat * alf_ref[...], hsel_ref[...],
                     preferred_element_type=jnp.float32)
        er = jnp.dot(feat * arf_ref[...], hsel_ref[...],
                     preferred_element_type=jnp.float32)
        z = jnp.zeros((bn, 16 - H), jnp.float32)
        l_ref[...] = jnp.concatenate([el, z], axis=1)
        r_ref[...] = jnp.concatenate([er, z], axis=1)

    return pl.pallas_call(
        body,
        grid=(n // bn,),
        in_specs=[
            pl.BlockSpec((bn, K), lambda i: (i, 0)),
            pl.BlockSpec((K, D), lambda i: (0, 0)),
            pl.BlockSpec((1, D), lambda i: (0, 0)),
            pl.BlockSpec((1, D), lambda i: (0, 0)),
            pl.BlockSpec((D, H), lambda i: (0, 0)),
        ],
        out_specs=[
            pl.BlockSpec((bn, D), lambda i: (i, 0)),
            pl.BlockSpec((bn, 16), lambda i: (i, 0)),
            pl.BlockSpec((bn, 16), lambda i: (i, 0)),
        ],
        out_shape=[
            jax.ShapeDtypeStruct((n, D), jnp.float32),
            jax.ShapeDtypeStruct((n, 16), jnp.float32),
            jax.ShapeDtypeStruct((n, 16), jnp.float32),
        ],
    )(x, W, alf, arf, hsel)


def _tc_stats(raw, den, hselT):
    n, D = raw.shape
    H = hselT.shape[0]

    def body(raw_ref, den_ref, hselT_ref, out_ref):
        i = pl.program_id(0)
        den2 = den_ref[0] + den_ref[1]
        denf = jnp.dot(den2[:, :H], hselT_ref[...],
                       preferred_element_type=jnp.float32)
        x = raw_ref[...] / jnp.maximum(denf, 1e-9)
        st = jnp.stack([jnp.sum(x, axis=0), jnp.sum(x * x, axis=0)])

        @pl.when(i == 0)
        def _():
            out_ref[...] = jnp.zeros_like(out_ref)
        out_ref[...] += st

    return pl.pallas_call(
        body,
        grid=(n // BN,),
        in_specs=[
            pl.BlockSpec((BN, D), lambda i: (i, 0)),
            pl.BlockSpec((2, BN, 16), lambda i: (0, i, 0)),
            pl.BlockSpec((H, D), lambda i: (0, 0)),
        ],
        out_specs=pl.BlockSpec((2, D), lambda i: (0, 0)),
        out_shape=jax.ShapeDtypeStruct((2, D), jnp.float32),
    )(raw, den, hselT)


def _tc_apply(raw, den, st, af, gf, bf, hselT, msel, rpw, rpb,
              W2=None, alf2=None, arf2=None, hsel2=None):
    n, D = raw.shape
    H = hselT.shape[0]
    with_next = W2 is not None

    def body(*refs):
        if with_next:
            (raw_ref, den_ref, st_ref, af_ref, gf_ref, bf_ref, hselT_ref,
             msel_ref, rpw_ref, rpb_ref, w2_ref, alf2_ref, arf2_ref, hsel2_ref,
             pooled_ref, feat2_ref, l2_ref, r2_ref) = refs
        else:
            (raw_ref, den_ref, st_ref, af_ref, gf_ref, bf_ref, hselT_ref,
             msel_ref, rpw_ref, rpb_ref, pooled_ref) = refs
        i = pl.program_id(0)
        den2 = den_ref[0] + den_ref[1]
        denf = jnp.dot(den2[:, :H], hselT_ref[...],
                       preferred_element_type=jnp.float32)
        x = raw_ref[...] / jnp.maximum(denf, 1e-9)
        st_v = st_ref[...]
        m = st_v[0:1] * (1.0 / N)
        e2 = st_v[1:2] * (1.0 / N)
        a = af_ref[...]
        var = e2 - (2.0 * a - a * a) * m * m
        xn = gf_ref[...] * (x - a * m) * jax.lax.rsqrt(var + EPS) + bf_ref[...]
        h = jnp.where(xn > 0, xn, 0.01 * xn)
        gmean = jnp.dot(h, msel_ref[...], preferred_element_type=jnp.float32) * (1.0 / H)
        phi = jnp.maximum(
            jnp.dot(gmean, rpw_ref[...], preferred_element_type=jnp.float32)
            + rpb_ref[...], 0.0)
        rowid = i * BN + lax.broadcasted_iota(jnp.int32, (BN, 1), 0)
        phi = jnp.where(rowid < N, phi, 0.0)

        @pl.when(i == 0)
        def _():
            pooled_ref[...] = jnp.zeros_like(pooled_ref)
        pooled_ref[...] += jnp.sum(phi, axis=0, keepdims=True)

        if with_next:
            feat2 = jnp.dot(h, w2_ref[...], preferred_element_type=jnp.float32)
            feat2_ref[...] = feat2
            H2 = hsel2_ref.shape[1]
            el2 = jnp.dot(feat2 * alf2_ref[...], hsel2_ref[...],
                          preferred_element_type=jnp.float32)
            er2 = jnp.dot(feat2 * arf2_ref[...], hsel2_ref[...],
                          preferred_element_type=jnp.float32)
            z = jnp.zeros((BN, 16 - H2), jnp.float32)
            l2_ref[...] = jnp.concatenate([el2, z], axis=1)
            r2_ref[...] = jnp.concatenate([er2, z], axis=1)

    in_specs = [
        pl.BlockSpec((BN, D), lambda i: (i, 0)),
        pl.BlockSpec((2, BN, 16), lambda i: (0, i, 0)),
        pl.BlockSpec((2, D), lambda i: (0, 0)),
        pl.BlockSpec((1, D), lambda i: (0, 0)),
        pl.BlockSpec((1, D), lambda i: (0, 0)),
        pl.BlockSpec((1, D), lambda i: (0, 0)),
        pl.BlockSpec((H, D), lambda i: (0, 0)),
        pl.BlockSpec((D, 256), lambda i: (0, 0)),
        pl.BlockSpec((256, 512), lambda i: (0, 0)),
        pl.BlockSpec((1, 512), lambda i: (0, 0)),
    ]
    out_specs = [pl.BlockSpec((1, 512), lambda i: (0, 0))]
    out_shape = [jax.ShapeDtypeStruct((1, 512), jnp.float32)]
    args = [raw, den, st, af, gf, bf, hselT, msel, rpw, rpb]
    if with_next:
        D2 = W2.shape[1]
        H2 = hsel2.shape[1]
        in_specs += [
            pl.BlockSpec((D, D2), lambda i: (0, 0)),
            pl.BlockSpec((1, D2), lambda i: (0, 0)),
            pl.BlockSpec((1, D2), lambda i: (0, 0)),
            pl.BlockSpec((D2, H2), lambda i: (0, 0)),
        ]
        out_specs += [
            pl.BlockSpec((BN, D2), lambda i: (i, 0)),
            pl.BlockSpec((BN, 16), lambda i: (i, 0)),
            pl.BlockSpec((BN, 16), lambda i: (i, 0)),
        ]
        out_shape += [
            jax.ShapeDtypeStruct((n, D2), jnp.float32),
            jax.ShapeDtypeStruct((n, 16), jnp.float32),
            jax.ShapeDtypeStruct((n, 16), jnp.float32),
        ]
        args += [W2, alf2, arf2, hsel2]

    return pl.pallas_call(
        body,
        grid=(n // BN,),
        in_specs=in_specs,
        out_specs=out_specs,
        out_shape=out_shape,
    )(*args)


def _tc_final(p1, p2, r1w, r1b, r2w, r2b, c1w, c1b, c2w, c2b):
    def body(p1_ref, p2_ref, r1w_ref, r1b_ref, r2w_ref, r2b_ref,
             c1w_ref, c1b_ref, c2w_ref, c2b_ref, out_ref):
        r1 = jnp.dot(p1_ref[...], r1w_ref[...],
                     preferred_element_type=jnp.float32) + r1b_ref[...]
        r2 = jnp.dot(p2_ref[...], r2w_ref[...],
                     preferred_element_type=jnp.float32) + r2b_ref[...]
        ro = jnp.concatenate([r1, r2], axis=1)
        hc = jnp.maximum(
            jnp.dot(ro, c1w_ref[...], preferred_element_type=jnp.float32)
            + c1b_ref[...], 0.0)
        out_ref[...] = jnp.dot(hc, c2w_ref[...],
                               preferred_element_type=jnp.float32) + c2b_ref[...]

    return pl.pallas_call(
        body,
        out_shape=jax.ShapeDtypeStruct((1, 10), jnp.float32),
    )(p1, p2, r1w, r1b, r2w, r2b, c1w, c1b, c2w, c2b)


def kernel(features, edge_index, W1, al1, ar1, gn1_alpha, gn1_gamma, gn1_beta,
           W2, al2, ar2, gn2_alpha, gn2_gamma, gn2_beta,
           r1_pw, r1_pb, r1_rw, r1_rb, r2_pw, r2_pb, r2_rw, r2_rb,
           c1_w, c1_b, c2_w, c2_b):
    f32 = jnp.float32
    src = edge_index[0]
    dst = edge_index[1]

    hsel1 = jnp.repeat(jnp.eye(4, dtype=f32), 256, axis=0)
    hsel2 = jnp.repeat(jnp.eye(2, dtype=f32), 256, axis=0)
    msel1 = jnp.tile(jnp.eye(256, dtype=f32), (4, 1))
    msel2 = jnp.tile(jnp.eye(256, dtype=f32), (2, 1))

    alf1 = al1.reshape(1, -1)
    arf1 = ar1.reshape(1, -1)
    alf2 = al2.reshape(1, -1)
    arf2 = ar2.reshape(1, -1)
    af1 = jnp.tile(gn1_alpha, 4).reshape(1, -1)
    gf1 = jnp.tile(gn1_gamma, 4).reshape(1, -1)
    bf1 = jnp.tile(gn1_beta, 4).reshape(1, -1)
    af2 = jnp.tile(gn2_alpha, 2).reshape(1, -1)
    gf2 = jnp.tile(gn2_gamma, 2).reshape(1, -1)
    bf2 = jnp.tile(gn2_beta, 2).reshape(1, -1)

    zrows = jnp.zeros((RPT, 128), f32)
    zrows16 = jnp.zeros((RPT, 16), f32)

    feat1, ltab1, rtab1 = _tc_feat(features, W1, alf1, arf1, hsel1)
    ext2_1, den1 = _sc_edge_l1(ltab1, rtab1, src, dst, zrows16)
    extt1 = _tc_ext_transpose(ext2_1)
    (raw1,) = _sc_agg_l1(feat1.reshape(N * 8, 128), extt1, src, dst, zrows)
    raw1f = raw1.transpose(1, 0, 2).reshape(NPAD, 1024)
    st1 = _tc_stats(raw1f, den1, hsel1.T)
    pooled1, feat2, ltab2, rtab2 = _tc_apply(
        raw1f, den1, st1, af1, gf1, bf1, hsel1.T, msel1, r1_pw,
        r1_pb.reshape(1, -1), W2=W2, alf2=alf2, arf2=arf2, hsel2=hsel2)

    ext2_2, den2 = _sc_edge_l2(ltab2, rtab2, src, dst, zrows16)
    extt2 = _tc_ext_transpose(ext2_2)
    (raw2,) = _sc_agg_l2(feat2.reshape(NPAD * 4, 128), extt2, src, dst, zrows)
    raw2f = raw2.transpose(1, 0, 2).reshape(NPAD, 512)
    st2 = _tc_stats(raw2f, den2, hsel2.T)
    (pooled2,) = _tc_apply(raw2f, den2, st2, af2, gf2, bf2, hsel2.T, msel2,
                           r2_pw, r2_pb.reshape(1, -1))

    return _tc_final(pooled1, pooled2,
                     r1_rw, r1_rb.reshape(1, -1), r2_rw, r2_rb.reshape(1, -1),
                     c1_w, c1_b.reshape(1, -1), c2_w, c2_b.reshape(1, -1))


# speedup vs baseline: 14.0577x; 1.0214x over previous
import functools

import jax
import jax.numpy as jnp
from jax import lax
from jax.experimental import pallas as pl
from jax.experimental.pallas import tpu as pltpu
from jax.experimental.pallas import tpu_sc as plsc

N = 10000
E = 160000
EPS = 1e-5
BN = 1024
NPAD = 10240
CHUNK = 128
NCH = E // CHUNK
NS = 16
RPT = NPAD // NS


def _splat(val):
    return jnp.full((16,), val, jnp.int32)


def _make_sc_edge(H):
    mesh = plsc.VectorSubcoreMesh(core_axis_name="c", subcore_axis_name="s")
    out_type = (
        jax.ShapeDtypeStruct((E, 16), jnp.float32),
        jax.ShapeDtypeStruct((2, NPAD, 16), jnp.float32),
    )
    scratch_types = [
        pltpu.VMEM((CHUNK,), jnp.int32),
        pltpu.VMEM((CHUNK,), jnp.int32),
        pltpu.VMEM((CHUNK, 16), jnp.float32),
        pltpu.VMEM((CHUNK, 16), jnp.float32),
        pltpu.VMEM((CHUNK, 16), jnp.float32),
        pltpu.VMEM_SHARED((NPAD, 16), jnp.float32),
        pltpu.SemaphoreType.DMA,
    ]

    @functools.partial(
        pl.kernel, out_type=out_type, mesh=mesh, scratch_types=scratch_types,
        compiler_params=pltpu.CompilerParams(use_tc_tiling_on_sc=False))
    def sc_edge(ltab, rtab, srcv, dstv, zrows16, ext2, denom,
                srcb, dstb, elrows, errows, exrows, densh, sem):
        c = lax.axis_index("c")
        s = lax.axis_index("s")

        pltpu.sync_copy(zrows16, densh.at[pl.ds(s * RPT, RPT)])
        plsc.subcore_barrier()

        nch2 = (NCH // 2) // NS + jnp.where(s < (NCH // 2) % NS, 1, 0)

        def phase1(i, carry):
            ch = (s + NS * i) * 2 + c
            e0 = ch * CHUNK
            pltpu.sync_copy(srcv.at[pl.ds(e0, CHUNK)], srcb)
            pltpu.sync_copy(dstv.at[pl.ds(e0, CHUNK)], dstb)
            pltpu.async_copy(ltab.at[srcb], elrows, sem).wait()
            pltpu.async_copy(rtab.at[dstb], errows, sem).wait()

            def edge(e, carry2):
                a = elrows[e, pl.ds(0, 16)] + errows[e, pl.ds(0, 16)]
                v = jnp.exp(jnp.where(a > 0, a, 0.2 * a))
                exrows[e, pl.ds(0, 16)] = v
                return carry2
            lax.fori_loop(0, CHUNK, edge, 0)

            pltpu.sync_copy(exrows, densh.at[dstb], add=True)
            pltpu.sync_copy(exrows, ext2.at[pl.ds(e0, CHUNK)])
            return carry
        lax.fori_loop(0, nch2, phase1, 0)
        plsc.subcore_barrier()

        sl = pl.ds(s * RPT, RPT)
        pltpu.sync_copy(densh.at[sl], denom.at[c, sl])

    return sc_edge


def _make_sc_agg(Dg, H):
    GPC = Dg // 2
    mesh = plsc.VectorSubcoreMesh(core_axis_name="c", subcore_axis_name="s")
    out_type = (
        jax.ShapeDtypeStruct((Dg, NPAD, 128), jnp.float32),
    )
    scratch_types = [
        pltpu.VMEM((CHUNK,), jnp.int32),
        pltpu.VMEM((CHUNK,), jnp.int32),
        pltpu.VMEM((CHUNK,), jnp.int32),
        pltpu.VMEM((CHUNK + 16,), jnp.float32),
        pltpu.VMEM((CHUNK, 128), jnp.float32),
        pltpu.VMEM_SHARED((NPAD, 128), jnp.float32),
        pltpu.SemaphoreType.DMA,
    ]

    @functools.partial(pl.kernel, out_type=out_type, mesh=mesh,
                       scratch_types=scratch_types)
    def sc_agg(featv, extt, srcv, dstv, zrows, raw,
               srcb, dstb, gidxb, exv, rows, acc, sem):
        c = lax.axis_index("c")
        s = lax.axis_index("s")
        nch = NCH // NS + jnp.where(s < NCH % NS, 1, 0)

        for gi in range(GPC):
            g = c * GPC + gi
            h = g // 2

            pltpu.sync_copy(zrows, acc.at[pl.ds(s * RPT, RPT)])
            plsc.subcore_barrier()

            def phase2(i, carry):
                e0 = (s + NS * i) * CHUNK
                pltpu.sync_copy(srcv.at[pl.ds(e0, CHUNK)], srcb)
                pltpu.sync_copy(dstv.at[pl.ds(e0, CHUNK)], dstb)
                pltpu.sync_copy(extt.at[h, 0, pl.ds(e0, CHUNK)],
                                exv.at[pl.ds(0, CHUNK)])
                for j in range(CHUNK // 16):
                    sv = srcb[pl.ds(j * 16, 16)]
                    gidxb[pl.ds(j * 16, 16)] = sv * Dg + g
                pltpu.async_copy(featv.at[gidxb], rows, sem).wait()

                def edge(e, carry2):
                    ex_e = exv[pl.ds(e, 16)][0]
                    for j in range(8):
                        sl = pl.ds(j * 16, 16)
                        rows[e, sl] = rows[e, sl] * ex_e
                    return carry2
                lax.fori_loop(0, CHUNK, edge, 0, unroll=8)

                pltpu.sync_copy(rows, acc.at[dstb], add=True)
                return carry
            lax.fori_loop(0, nch, phase2, 0)
            plsc.subcore_barrier()

            sl = pl.ds(s * RPT, RPT)
            pltpu.sync_copy(acc.at[sl], raw.at[g, sl])
            plsc.subcore_barrier()

    return sc_agg


_sc_edge_l1 = _make_sc_edge(4)
_sc_edge_l2 = _make_sc_edge(2)
_sc_agg_l1 = _make_sc_agg(8, 4)
_sc_agg_l2 = _make_sc_agg(4, 2)


def _tc_ext_transpose(ext2):
    BE = 3200

    def body(x_ref, o_ref):
        o_ref[...] = x_ref[...].T.reshape(16, 1, BE)

    return pl.pallas_call(
        body,
        grid=(E // BE,),
        in_specs=[pl.BlockSpec((BE, 16), lambda i: (i, 0))],
        out_specs=pl.BlockSpec((16, 1, BE), lambda i: (0, 0, i)),
        out_shape=jax.ShapeDtypeStruct((16, 1, E), jnp.float32),
    )(ext2)


def _tc_feat(x, W, alf, arf, hsel):
    n, K = x.shape
    D = W.shape[1]
    H = hsel.shape[1]
    bn = 1000

    def body(x_ref, w_ref, alf_ref, arf_ref, hsel_ref, feat_ref, l_ref, r_ref):
        feat = jnp.dot(x_ref[...], w_ref[...], preferred_element_type=jnp.float32)
        feat_ref[...] = feat
        el = jnp.dot(feat * alf_ref[...], hsel_ref[...],
                     preferred_element_type=jnp.float32)
        er = jnp.dot(feat * arf_ref[...], hsel_ref[...],
                     preferred_element_type=jnp.float32)
        z = jnp.zeros((bn, 16 - H), jnp.float32)
        l_ref[...] = jnp.concatenate([el, z], axis=1)
        r_ref[...] = jnp.concatenate([er, z], axis=1)

    return pl.pallas_call(
        body,
        grid=(n // bn,),
        in_specs=[
            pl.BlockSpec((bn, K), lambda i: (i, 0)),
            pl.BlockSpec((K, D), lambda i: (0, 0)),
            pl.BlockSpec((1, D), lambda i: (0, 0)),
            pl.BlockSpec((1, D), lambda i: (0, 0)),
            pl.BlockSpec((D, H), lambda i: (0, 0)),
        ],
        out_specs=[
            pl.BlockSpec((bn, D), lambda i: (i, 0)),
            pl.BlockSpec((bn, 16), lambda i: (i, 0)),
            pl.BlockSpec((bn, 16), lambda i: (i, 0)),
        ],
        out_shape=[
            jax.ShapeDtypeStruct((n, D), jnp.float32),
            jax.ShapeDtypeStruct((n, 16), jnp.float32),
            jax.ShapeDtypeStruct((n, 16), jnp.float32),
        ],
    )(x, W, alf, arf, hsel)


def _tc_stats(raw, den, hselT):
    n, D = raw.shape
    H = hselT.shape[0]

    def body(raw_ref, den_ref, hselT_ref, out_ref):
        i = pl.program_id(0)
        den2 = den_ref[0] + den_ref[1]
        denf = jnp.dot(den2[:, :H], hselT_ref[...],
                       preferred_element_type=jnp.float32)
        x = raw_ref[...] / jnp.maximum(denf, 1e-9)
        st = jnp.stack([jnp.sum(x, axis=0), jnp.sum(x * x, axis=0)])

        @pl.when(i == 0)
        def _():
            out_ref[...] = jnp.zeros_like(out_ref)
        out_ref[...] += st

    return pl.pallas_call(
        body,
        grid=(n // BN,),
        in_specs=[
            pl.BlockSpec((BN, D), lambda i: (i, 0)),
            pl.BlockSpec((2, BN, 16), lambda i: (0, i, 0)),
            pl.BlockSpec((H, D), lambda i: (0, 0)),
        ],
        out_specs=pl.BlockSpec((2, D), lambda i: (0, 0)),
        out_shape=jax.ShapeDtypeStruct((2, D), jnp.float32),
    )(raw, den, hselT)


def _tc_apply(raw, den, st, af, gf, bf, hselT, msel, rpw, rpb,
              W2=None, alf2=None, arf2=None, hsel2=None):
    n, D = raw.shape
    H = hselT.shape[0]
    with_next = W2 is not None

    def body(*refs):
        if with_next:
            (raw_ref, den_ref, st_ref, af_ref, gf_ref, bf_ref, hselT_ref,
             msel_ref, rpw_ref, rpb_ref, w2_ref, alf2_ref, arf2_ref, hsel2_ref,
             pooled_ref, feat2_ref, l2_ref, r2_ref) = refs
        else:
            (raw_ref, den_ref, st_ref, af_ref, gf_ref, bf_ref, hselT_ref,
             msel_ref, rpw_ref, rpb_ref, pooled_ref) = refs
        i = pl.program_id(0)
        den2 = den_ref[0] + den_ref[1]
        denf = jnp.dot(den2[:, :H], hselT_ref[...],
                       preferred_element_type=jnp.float32)
        x = raw_ref[...] / jnp.maximum(denf, 1e-9)
        st_v = st_ref[...]
        m = st_v[0:1] * (1.0 / N)
        e2 = st_v[1:2] * (1.0 / N)
        a = af_ref[...]
        var = e2 - (2.0 * a - a * a) * m * m
        xn = gf_ref[...] * (x - a * m) * jax.lax.rsqrt(var + EPS) + bf_ref[...]
        h = jnp.where(xn > 0, xn, 0.01 * xn)
        gmean = jnp.dot(h, msel_ref[...], preferred_element_type=jnp.float32) * (1.0 / H)
        phi = jnp.maximum(
            jnp.dot(gmean, rpw_ref[...], preferred_element_type=jnp.float32)
            + rpb_ref[...], 0.0)
        rowid = i * BN + lax.broadcasted_iota(jnp.int32, (BN, 1), 0)
        phi = jnp.where(rowid < N, phi, 0.0)

        @pl.when(i == 0)
        def _():
            pooled_ref[...] = jnp.zeros_like(pooled_ref)
        pooled_ref[...] += jnp.sum(phi, axis=0, keepdims=True)

        if with_next:
            feat2 = jnp.dot(h, w2_ref[...], preferred_element_type=jnp.float32)
            feat2_ref[...] = feat2
            H2 = hsel2_ref.shape[1]
            el2 = jnp.dot(feat2 * alf2_ref[...], hsel2_ref[...],
                          preferred_element_type=jnp.float32)
            er2 = jnp.dot(feat2 * arf2_ref[...], hsel2_ref[...],
                          preferred_element_type=jnp.float32)
            z = jnp.zeros((BN, 16 - H2), jnp.float32)
            l2_ref[...] = jnp.concatenate([el2, z], axis=1)
            r2_ref[...] = jnp.concatenate([er2, z], axis=1)

    in_specs = [
        pl.BlockSpec((BN, D), lambda i: (i, 0)),
        pl.BlockSpec((2, BN, 16), lambda i: (0, i, 0)),
        pl.BlockSpec((2, D), lambda i: (0, 0)),
        pl.BlockSpec((1, D), lambda i: (0, 0)),
        pl.BlockSpec((1, D), lambda i: (0, 0)),
        pl.BlockSpec((1, D), lambda i: (0, 0)),
        pl.BlockSpec((H, D), lambda i: (0, 0)),
        pl.BlockSpec((D, 256), lambda i: (0, 0)),
        pl.BlockSpec((256, 512), lambda i: (0, 0)),
        pl.BlockSpec((1, 512), lambda i: (0, 0)),
    ]
    out_specs = [pl.BlockSpec((1, 512), lambda i: (0, 0))]
    out_shape = [jax.ShapeDtypeStruct((1, 512), jnp.float32)]
    args = [raw, den, st, af, gf, bf, hselT, msel, rpw, rpb]
    if with_next:
        D2 = W2.shape[1]
        H2 = hsel2.shape[1]
        in_specs += [
            pl.BlockSpec((D, D2), lambda i: (0, 0)),
            pl.BlockSpec((1, D2), lambda i: (0, 0)),
            pl.BlockSpec((1, D2), lambda i: (0, 0)),
            pl.BlockSpec((D2, H2), lambda i: (0, 0)),
        ]
        out_specs += [
            pl.BlockSpec((BN, D2), lambda i: (i, 0)),
            pl.BlockSpec((BN, 16), lambda i: (i, 0)),
            pl.BlockSpec((BN, 16), lambda i: (i, 0)),
        ]
        out_shape += [
            jax.ShapeDtypeStruct((n, D2), jnp.float32),
            jax.ShapeDtypeStruct((n, 16), jnp.float32),
            jax.ShapeDtypeStruct((n, 16), jnp.float32),
        ]
        args += [W2, alf2, arf2, hsel2]

    return pl.pallas_call(
        body,
        grid=(n // BN,),
        in_specs=in_specs,
        out_specs=out_specs,
        out_shape=out_shape,
    )(*args)


def _tc_final(p1, p2, r1w, r1b, r2w, r2b, c1w, c1b, c2w, c2b):
    def body(p1_ref, p2_ref, r1w_ref, r1b_ref, r2w_ref, r2b_ref,
             c1w_ref, c1b_ref, c2w_ref, c2b_ref, out_ref):
        r1 = jnp.dot(p1_ref[...], r1w_ref[...],
                     preferred_element_type=jnp.float32) + r1b_ref[...]
        r2 = jnp.dot(p2_ref[...], r2w_ref[...],
                     preferred_element_type=jnp.float32) + r2b_ref[...]
        ro = jnp.concatenate([r1, r2], axis=1)
        hc = jnp.maximum(
            jnp.dot(ro, c1w_ref[...], preferred_element_type=jnp.float32)
            + c1b_ref[...], 0.0)
        out_ref[...] = jnp.dot(hc, c2w_ref[...],
                               preferred_element_type=jnp.float32) + c2b_ref[...]

    return pl.pallas_call(
        body,
        out_shape=jax.ShapeDtypeStruct((1, 10), jnp.float32),
    )(p1, p2, r1w, r1b, r2w, r2b, c1w, c1b, c2w, c2b)


def kernel(features, edge_index, W1, al1, ar1, gn1_alpha, gn1_gamma, gn1_beta,
           W2, al2, ar2, gn2_alpha, gn2_gamma, gn2_beta,
           r1_pw, r1_pb, r1_rw, r1_rb, r2_pw, r2_pb, r2_rw, r2_rb,
           c1_w, c1_b, c2_w, c2_b):
    f32 = jnp.float32
    src = edge_index[0]
    dst = edge_index[1]

    hsel1 = jnp.repeat(jnp.eye(4, dtype=f32), 256, axis=0)
    hsel2 = jnp.repeat(jnp.eye(2, dtype=f32), 256, axis=0)
    msel1 = jnp.tile(jnp.eye(256, dtype=f32), (4, 1))
    msel2 = jnp.tile(jnp.eye(256, dtype=f32), (2, 1))

    alf1 = al1.reshape(1, -1)
    arf1 = ar1.reshape(1, -1)
    alf2 = al2.reshape(1, -1)
    arf2 = ar2.reshape(1, -1)
    af1 = jnp.tile(gn1_alpha, 4).reshape(1, -1)
    gf1 = jnp.tile(gn1_gamma, 4).reshape(1, -1)
    bf1 = jnp.tile(gn1_beta, 4).reshape(1, -1)
    af2 = jnp.tile(gn2_alpha, 2).reshape(1, -1)
    gf2 = jnp.tile(gn2_gamma, 2).reshape(1, -1)
    bf2 = jnp.tile(gn2_beta, 2).reshape(1, -1)

    zrows = jnp.zeros((RPT, 128), f32)
    zrows16 = jnp.zeros((RPT, 16), f32)

    feat1, ltab1, rtab1 = _tc_feat(features, W1, alf1, arf1, hsel1)
    ext2_1, den1 = _sc_edge_l1(ltab1, rtab1, src, dst, zrows16)
    extt1 = _tc_ext_transpose(ext2_1)
    (raw1,) = _sc_agg_l1(feat1.reshape(N * 8, 128), extt1, src, dst, zrows)
    raw1f = raw1.transpose(1, 0, 2).reshape(NPAD, 1024)
    st1 = _tc_stats(raw1f, den1, hsel1.T)
    pooled1, feat2, ltab2, rtab2 = _tc_apply(
        raw1f, den1, st1, af1, gf1, bf1, hsel1.T, msel1, r1_pw,
        r1_pb.reshape(1, -1), W2=W2, alf2=alf2, arf2=arf2, hsel2=hsel2)

    ext2_2, den2 = _sc_edge_l2(ltab2, rtab2, src, dst, zrows16)
    extt2 = _tc_ext_transpose(ext2_2)
    (raw2,) = _sc_agg_l2(feat2.reshape(NPAD * 4, 128), extt2, src, dst, zrows)
    raw2f = raw2.transpose(1, 0, 2).reshape(NPAD, 512)
    st2 = _tc_stats(raw2f, den2, hsel2.T)
    (pooled2,) = _tc_apply(raw2f, den2, st2, af2, gf2, bf2, hsel2.T, msel2,
                           r2_pw, r2_pb.reshape(1, -1))

    return _tc_final(pooled1, pooled2,
                     r1_rw, r1_rb.reshape(1, -1), r2_rw, r2_rb.reshape(1, -1),
                     c1_w, c1_b.reshape(1, -1), c2_w, c2_b.reshape(1, -1))


# speedup vs baseline: 20.4894x; 1.4575x over previous
import functools

import jax
import jax.numpy as jnp
from jax import lax
from jax.experimental import pallas as pl
from jax.experimental.pallas import tpu as pltpu
from jax.experimental.pallas import tpu_sc as plsc

N = 10000
E = 160000
EPS = 1e-5
BN = 1024
NPAD = 10240
CHUNK = 128
NCH = E // CHUNK
NS = 16
RPT = NPAD // NS


def _splat(val):
    return jnp.full((16,), val, jnp.int32)


def _make_sc_edge(H):
    mesh = plsc.VectorSubcoreMesh(core_axis_name="c", subcore_axis_name="s")
    out_type = (
        jax.ShapeDtypeStruct((E, 16), jnp.float32),
        jax.ShapeDtypeStruct((2, NPAD, 16), jnp.float32),
    )
    scratch_types = [
        pltpu.VMEM((CHUNK,), jnp.int32),
        pltpu.VMEM((CHUNK,), jnp.int32),
        pltpu.VMEM((CHUNK, 16), jnp.float32),
        pltpu.VMEM((CHUNK, 16), jnp.float32),
        pltpu.VMEM((CHUNK, 16), jnp.float32),
        pltpu.VMEM_SHARED((NPAD, 16), jnp.float32),
        pltpu.SemaphoreType.DMA,
    ]

    @functools.partial(
        pl.kernel, out_type=out_type, mesh=mesh, scratch_types=scratch_types,
        compiler_params=pltpu.CompilerParams(use_tc_tiling_on_sc=False))
    def sc_edge(ltab, rtab, srcv, dstv, zrows16, ext2, denom,
                srcb, dstb, elrows, errows, exrows, densh, sem):
        c = lax.axis_index("c")
        s = lax.axis_index("s")

        pltpu.sync_copy(zrows16, densh.at[pl.ds(s * RPT, RPT)])
        plsc.subcore_barrier()

        nch2 = (NCH // 2) // NS + jnp.where(s < (NCH // 2) % NS, 1, 0)

        def phase1(i, carry):
            ch = (s + NS * i) * 2 + c
            e0 = ch * CHUNK
            pltpu.sync_copy(srcv.at[pl.ds(e0, CHUNK)], srcb)
            pltpu.sync_copy(dstv.at[pl.ds(e0, CHUNK)], dstb)
            pltpu.async_copy(ltab.at[srcb], elrows, sem).wait()
            pltpu.async_copy(rtab.at[dstb], errows, sem).wait()

            def edge(e, carry2):
                a = elrows[e, pl.ds(0, 16)] + errows[e, pl.ds(0, 16)]
                v = jnp.exp(jnp.where(a > 0, a, 0.2 * a))
                exrows[e, pl.ds(0, 16)] = v
                return carry2
            lax.fori_loop(0, CHUNK, edge, 0)

            pltpu.sync_copy(exrows, densh.at[dstb], add=True)
            pltpu.sync_copy(exrows, ext2.at[pl.ds(e0, CHUNK)])
            return carry
        lax.fori_loop(0, nch2, phase1, 0)
        plsc.subcore_barrier()

        sl = pl.ds(s * RPT, RPT)
        pltpu.sync_copy(densh.at[sl], denom.at[c, sl])

    return sc_edge


def _make_sc_agg(Dg, H):
    GPC = Dg // 2
    mesh = plsc.VectorSubcoreMesh(core_axis_name="c", subcore_axis_name="s")
    out_type = (
        jax.ShapeDtypeStruct((Dg, NPAD, 128), jnp.float32),
    )
    scratch_types = [
        pltpu.VMEM((CHUNK,), jnp.int32),
        pltpu.VMEM((CHUNK,), jnp.int32),
        pltpu.VMEM((CHUNK,), jnp.int32),
        pltpu.VMEM((CHUNK + 16,), jnp.float32),
        pltpu.VMEM((CHUNK, 128), jnp.float32),
        pltpu.VMEM((CHUNK,), jnp.int32),
        pltpu.VMEM((CHUNK,), jnp.int32),
        pltpu.VMEM((CHUNK,), jnp.int32),
        pltpu.VMEM((CHUNK + 16,), jnp.float32),
        pltpu.VMEM((CHUNK, 128), jnp.float32),
        pltpu.VMEM_SHARED((NPAD, 128), jnp.float32),
        pltpu.SemaphoreType.DMA,
        pltpu.SemaphoreType.DMA,
        pltpu.SemaphoreType.DMA,
    ]

    @functools.partial(pl.kernel, out_type=out_type, mesh=mesh,
                       scratch_types=scratch_types)
    def sc_agg(featv, extt, srcv, dstv, zrows, raw,
               srcbA, dstbA, gidxbA, exvA, rowsA,
               srcbB, dstbB, gidxbB, exvB, rowsB,
               acc, semA, semB, semS):
        c = lax.axis_index("c")
        s = lax.axis_index("s")

        def stage_in(e0, srcb, dstb, exv, h, sem):
            d1 = pltpu.async_copy(srcv.at[pl.ds(e0, CHUNK)], srcb, sem)
            d2 = pltpu.async_copy(dstv.at[pl.ds(e0, CHUNK)], dstb, sem)
            d3 = pltpu.async_copy(extt.at[h, 0, pl.ds(e0, CHUNK)],
                                  exv.at[pl.ds(0, CHUNK)], sem)
            return d1, d2, d3

        def build_gidx(srcb, gidxb, g):
            for j in range(CHUNK // 16):
                sv = srcb[pl.ds(j * 16, 16)]
                gidxb[pl.ds(j * 16, 16)] = sv * Dg + g

        def scale(rows, exv):
            def edge(e, carry2):
                ex_e = exv[pl.ds(e, 16)][0]
                for j in range(8):
                    sl = pl.ds(j * 16, 16)
                    rows[e, sl] = rows[e, sl] * ex_e
                return carry2
            lax.fori_loop(0, CHUNK, edge, 0, unroll=8)

        for gi in range(GPC):
            g = c * GPC + gi
            h = g // 2

            pltpu.sync_copy(zrows, acc.at[pl.ds(s * RPT, RPT)])
            plsc.subcore_barrier()

            def pair(i, carry):
                e0A = (s + NS * (2 * i)) * CHUNK
                e0B = (s + NS * (2 * i + 1)) * CHUNK
                a1, a2, a3 = stage_in(e0A, srcbA, dstbA, exvA, h, semA)
                b1, b2, b3 = stage_in(e0B, srcbB, dstbB, exvB, h, semB)
                a1.wait(); a2.wait(); a3.wait()
                build_gidx(srcbA, gidxbA, g)
                gA = pltpu.async_copy(featv.at[gidxbA], rowsA, semA)
                b1.wait(); b2.wait(); b3.wait()
                build_gidx(srcbB, gidxbB, g)
                gA.wait()
                gB = pltpu.async_copy(featv.at[gidxbB], rowsB, semB)
                scale(rowsA, exvA)
                sA = pltpu.async_copy(rowsA, acc.at[dstbA], semS, add=True)
                gB.wait()
                scale(rowsB, exvB)
                sB = pltpu.async_copy(rowsB, acc.at[dstbB], semS, add=True)
                sA.wait()
                sB.wait()
                return carry
            lax.fori_loop(0, NCH // NS // 2, pair, 0)

            @pl.when(s < NCH % NS)
            def _():
                e0 = (s + NS * (NCH // NS)) * CHUNK
                a1, a2, a3 = stage_in(e0, srcbA, dstbA, exvA, h, semA)
                a1.wait(); a2.wait(); a3.wait()
                build_gidx(srcbA, gidxbA, g)
                pltpu.async_copy(featv.at[gidxbA], rowsA, semA).wait()
                scale(rowsA, exvA)
                pltpu.sync_copy(rowsA, acc.at[dstbA], add=True)

            plsc.subcore_barrier()
            sl = pl.ds(s * RPT, RPT)
            pltpu.sync_copy(acc.at[sl], raw.at[g, sl])
            plsc.subcore_barrier()

    return sc_agg


_sc_edge_l1 = _make_sc_edge(4)
_sc_edge_l2 = _make_sc_edge(2)
_sc_agg_l1 = _make_sc_agg(8, 4)
_sc_agg_l2 = _make_sc_agg(4, 2)


def _tc_ext_transpose(ext2):
    BE = 3200

    def body(x_ref, o_ref):
        o_ref[...] = x_ref[...].T.reshape(16, 1, BE)

    return pl.pallas_call(
        body,
        grid=(E // BE,),
        in_specs=[pl.BlockSpec((BE, 16), lambda i: (i, 0))],
        out_specs=pl.BlockSpec((16, 1, BE), lambda i: (0, 0, i)),
        out_shape=jax.ShapeDtypeStruct((16, 1, E), jnp.float32),
    )(ext2)


def _tc_feat(x, W, alf, arf, hsel):
    n, K = x.shape
    D = W.shape[1]
    H = hsel.shape[1]
    bn = 1000

    def body(x_ref, w_ref, alf_ref, arf_ref, hsel_ref, feat_ref, l_ref, r_ref):
        feat = jnp.dot(x_ref[...], w_ref[...], preferred_element_type=jnp.float32)
        feat_ref[...] = feat
        el = jnp.dot(feat * alf_ref[...], hsel_ref[...],
                     preferred_element_type=jnp.float32)
        er = jnp.dot(feat * arf_ref[...], hsel_ref[...],
                     preferred_element_type=jnp.float32)
        z = jnp.zeros((bn, 16 - H), jnp.float32)
        l_ref[...] = jnp.concatenate([el, z], axis=1)
        r_ref[...] = jnp.concatenate([er, z], axis=1)

    return pl.pallas_call(
        body,
        grid=(n // bn,),
        in_specs=[
            pl.BlockSpec((bn, K), lambda i: (i, 0)),
            pl.BlockSpec((K, D), lambda i: (0, 0)),
            pl.BlockSpec((1, D), lambda i: (0, 0)),
            pl.BlockSpec((1, D), lambda i: (0, 0)),
            pl.BlockSpec((D, H), lambda i: (0, 0)),
        ],
        out_specs=[
            pl.BlockSpec((bn, D), lambda i: (i, 0)),
            pl.BlockSpec((bn, 16), lambda i: (i, 0)),
            pl.BlockSpec((bn, 16), lambda i: (i, 0)),
        ],
        out_shape=[
            jax.ShapeDtypeStruct((n, D), jnp.float32),
            jax.ShapeDtypeStruct((n, 16), jnp.float32),
            jax.ShapeDtypeStruct((n, 16), jnp.float32),
        ],
    )(x, W, alf, arf, hsel)


def _tc_stats(raw, den, hselT):
    n, D = raw.shape
    H = hselT.shape[0]

    def body(raw_ref, den_ref, hselT_ref, out_ref):
        i = pl.program_id(0)
        den2 = den_ref[0] + den_ref[1]
        denf = jnp.dot(den2[:, :H], hselT_ref[...],
                       preferred_element_type=jnp.float32)
        x = raw_ref[...] / jnp.maximum(denf, 1e-9)
        st = jnp.stack([jnp.sum(x, axis=0), jnp.sum(x * x, axis=0)])

        @pl.when(i == 0)
        def _():
            out_ref[...] = jnp.zeros_like(out_ref)
        out_ref[...] += st

    return pl.pallas_call(
        body,
        grid=(n // BN,),
        in_specs=[
            pl.BlockSpec((BN, D), lambda i: (i, 0)),
            pl.BlockSpec((2, BN, 16), lambda i: (0, i, 0)),
            pl.BlockSpec((H, D), lambda i: (0, 0)),
        ],
        out_specs=pl.BlockSpec((2, D), lambda i: (0, 0)),
        out_shape=jax.ShapeDtypeStruct((2, D), jnp.float32),
    )(raw, den, hselT)


def _tc_apply(raw, den, st, af, gf, bf, hselT, msel, rpw, rpb,
              W2=None, alf2=None, arf2=None, hsel2=None):
    n, D = raw.shape
    H = hselT.shape[0]
    with_next = W2 is not None

    def body(*refs):
        if with_next:
            (raw_ref, den_ref, st_ref, af_ref, gf_ref, bf_ref, hselT_ref,
             msel_ref, rpw_ref, rpb_ref, w2_ref, alf2_ref, arf2_ref, hsel2_ref,
             pooled_ref, feat2_ref, l2_ref, r2_ref) = refs
        else:
            (raw_ref, den_ref, st_ref, af_ref, gf_ref, bf_ref, hselT_ref,
             msel_ref, rpw_ref, rpb_ref, pooled_ref) = refs
        i = pl.program_id(0)
        den2 = den_ref[0] + den_ref[1]
        denf = jnp.dot(den2[:, :H], hselT_ref[...],
                       preferred_element_type=jnp.float32)
        x = raw_ref[...] / jnp.maximum(denf, 1e-9)
        st_v = st_ref[...]
        m = st_v[0:1] * (1.0 / N)
        e2 = st_v[1:2] * (1.0 / N)
        a = af_ref[...]
        var = e2 - (2.0 * a - a * a) * m * m
        xn = gf_ref[...] * (x - a * m) * jax.lax.rsqrt(var + EPS) + bf_ref[...]
        h = jnp.where(xn > 0, xn, 0.01 * xn)
        gmean = jnp.dot(h, msel_ref[...], preferred_element_type=jnp.float32) * (1.0 / H)
        phi = jnp.maximum(
            jnp.dot(gmean, rpw_ref[...], preferred_element_type=jnp.float32)
            + rpb_ref[...], 0.0)
        rowid = i * BN + lax.broadcasted_iota(jnp.int32, (BN, 1), 0)
        phi = jnp.where(rowid < N, phi, 0.0)

        @pl.when(i == 0)
        def _():
            pooled_ref[...] = jnp.zeros_like(pooled_ref)
        pooled_ref[...] += jnp.sum(phi, axis=0, keepdims=True)

        if with_next:
            feat2 = jnp.dot(h, w2_ref[...], preferred_element_type=jnp.float32)
            feat2_ref[...] = feat2
            H2 = hsel2_ref.shape[1]
            el2 = jnp.dot(feat2 * alf2_ref[...], hsel2_ref[...],
                          preferred_element_type=jnp.float32)
            er2 = jnp.dot(feat2 * arf2_ref[...], hsel2_ref[...],
                          preferred_element_type=jnp.float32)
            z = jnp.zeros((BN, 16 - H2), jnp.float32)
            l2_ref[...] = jnp.concatenate([el2, z], axis=1)
            r2_ref[...] = jnp.concatenate([er2, z], axis=1)

    in_specs = [
        pl.BlockSpec((BN, D), lambda i: (i, 0)),
        pl.BlockSpec((2, BN, 16), lambda i: (0, i, 0)),
        pl.BlockSpec((2, D), lambda i: (0, 0)),
        pl.BlockSpec((1, D), lambda i: (0, 0)),
        pl.BlockSpec((1, D), lambda i: (0, 0)),
        pl.BlockSpec((1, D), lambda i: (0, 0)),
        pl.BlockSpec((H, D), lambda i: (0, 0)),
        pl.BlockSpec((D, 256), lambda i: (0, 0)),
        pl.BlockSpec((256, 512), lambda i: (0, 0)),
        pl.BlockSpec((1, 512), lambda i: (0, 0)),
    ]
    out_specs = [pl.BlockSpec((1, 512), lambda i: (0, 0))]
    out_shape = [jax.ShapeDtypeStruct((1, 512), jnp.float32)]
    args = [raw, den, st, af, gf, bf, hselT, msel, rpw, rpb]
    if with_next:
        D2 = W2.shape[1]
        H2 = hsel2.shape[1]
        in_specs += [
            pl.BlockSpec((D, D2), lambda i: (0, 0)),
            pl.BlockSpec((1, D2), lambda i: (0, 0)),
            pl.BlockSpec((1, D2), lambda i: (0, 0)),
            pl.BlockSpec((D2, H2), lambda i: (0, 0)),
        ]
        out_specs += [
            pl.BlockSpec((BN, D2), lambda i: (i, 0)),
            pl.BlockSpec((BN, 16), lambda i: (i, 0)),
            pl.BlockSpec((BN, 16), lambda i: (i, 0)),
        ]
        out_shape += [
            jax.ShapeDtypeStruct((n, D2), jnp.float32),
            jax.ShapeDtypeStruct((n, 16), jnp.float32),
            jax.ShapeDtypeStruct((n, 16), jnp.float32),
        ]
        args += [W2, alf2, arf2, hsel2]

    return pl.pallas_call(
        body,
        grid=(n // BN,),
        in_specs=in_specs,
        out_specs=out_specs,
        out_shape=out_shape,
    )(*args)


def _tc_final(p1, p2, r1w, r1b, r2w, r2b, c1w, c1b, c2w, c2b):
    def body(p1_ref, p2_ref, r1w_ref, r1b_ref, r2w_ref, r2b_ref,
             c1w_ref, c1b_ref, c2w_ref, c2b_ref, out_ref):
        r1 = jnp.dot(p1_ref[...], r1w_ref[...],
                     preferred_element_type=jnp.float32) + r1b_ref[...]
        r2 = jnp.dot(p2_ref[...], r2w_ref[...],
                     preferred_element_type=jnp.float32) + r2b_ref[...]
        ro = jnp.concatenate([r1, r2], axis=1)
        hc = jnp.maximum(
            jnp.dot(ro, c1w_ref[...], preferred_element_type=jnp.float32)
            + c1b_ref[...], 0.0)
        out_ref[...] = jnp.dot(hc, c2w_ref[...],
                               preferred_element_type=jnp.float32) + c2b_ref[...]

    return pl.pallas_call(
        body,
        out_shape=jax.ShapeDtypeStruct((1, 10), jnp.float32),
    )(p1, p2, r1w, r1b, r2w, r2b, c1w, c1b, c2w, c2b)


def kernel(features, edge_index, W1, al1, ar1, gn1_alpha, gn1_gamma, gn1_beta,
           W2, al2, ar2, gn2_alpha, gn2_gamma, gn2_beta,
           r1_pw, r1_pb, r1_rw, r1_rb, r2_pw, r2_pb, r2_rw, r2_rb,
           c1_w, c1_b, c2_w, c2_b):
    f32 = jnp.float32
    src = edge_index[0]
    dst = edge_index[1]

    hsel1 = jnp.repeat(jnp.eye(4, dtype=f32), 256, axis=0)
    hsel2 = jnp.repeat(jnp.eye(2, dtype=f32), 256, axis=0)
    msel1 = jnp.tile(jnp.eye(256, dtype=f32), (4, 1))
    msel2 = jnp.tile(jnp.eye(256, dtype=f32), (2, 1))

    alf1 = al1.reshape(1, -1)
    arf1 = ar1.reshape(1, -1)
    alf2 = al2.reshape(1, -1)
    arf2 = ar2.reshape(1, -1)
    af1 = jnp.tile(gn1_alpha, 4).reshape(1, -1)
    gf1 = jnp.tile(gn1_gamma, 4).reshape(1, -1)
    bf1 = jnp.tile(gn1_beta, 4).reshape(1, -1)
    af2 = jnp.tile(gn2_alpha, 2).reshape(1, -1)
    gf2 = jnp.tile(gn2_gamma, 2).reshape(1, -1)
    bf2 = jnp.tile(gn2_beta, 2).reshape(1, -1)

    zrows = jnp.zeros((RPT, 128), f32)
    zrows16 = jnp.zeros((RPT, 16), f32)

    feat1, ltab1, rtab1 = _tc_feat(features, W1, alf1, arf1, hsel1)
    ext2_1, den1 = _sc_edge_l1(ltab1, rtab1, src, dst, zrows16)
    extt1 = _tc_ext_transpose(ext2_1)
    (raw1,) = _sc_agg_l1(feat1.reshape(N * 8, 128), extt1, src, dst, zrows)
    raw1f = raw1.transpose(1, 0, 2).reshape(NPAD, 1024)
    st1 = _tc_stats(raw1f, den1, hsel1.T)
    pooled1, feat2, ltab2, rtab2 = _tc_apply(
        raw1f, den1, st1, af1, gf1, bf1, hsel1.T, msel1, r1_pw,
        r1_pb.reshape(1, -1), W2=W2, alf2=alf2, arf2=arf2, hsel2=hsel2)

    ext2_2, den2 = _sc_edge_l2(ltab2, rtab2, src, dst, zrows16)
    extt2 = _tc_ext_transpose(ext2_2)
    (raw2,) = _sc_agg_l2(feat2.reshape(NPAD * 4, 128), extt2, src, dst, zrows)
    raw2f = raw2.transpose(1, 0, 2).reshape(NPAD, 512)
    st2 = _tc_stats(raw2f, den2, hsel2.T)
    (pooled2,) = _tc_apply(raw2f, den2, st2, af2, gf2, bf2, hsel2.T, msel2,
                           r2_pw, r2_pb.reshape(1, -1))

    return _tc_final(pooled1, pooled2,
                     r1_rw, r1_rb.reshape(1, -1), r2_rw, r2_rb.reshape(1, -1),
                     c1_w, c1_b.reshape(1, -1), c2_w, c2_b.reshape(1, -1))
